# baseline (speedup 1.0000x reference)
import functools

import jax
import jax.numpy as jnp
from jax import lax
from jax.experimental import pallas as pl
from jax.experimental.pallas import tpu as pltpu
from jax.experimental.pallas import tpu_sc as plsc

SCALE = 0.7071067811865476
EMB = 64
SEQ = 200
PAD = 0
LANES = 16


def _sc_embed(tok_flat, word_table_scaled, pos_table):
    N = tok_flat.shape[0]
    info = plsc.get_sparse_core_info()
    NW = info.num_cores * info.num_subcores
    per_w = N // NW
    CH = SEQ
    NB = 6
    AHEAD = 4
    n_chunks = per_w // CH
    assert per_w % CH == 0
    KV = EMB // LANES

    mesh = plsc.VectorSubcoreMesh(core_axis_name="c", subcore_axis_name="s")

    @functools.partial(
        pl.kernel,
        out_type=jax.ShapeDtypeStruct((N, EMB), jnp.float32),
        mesh=mesh,
        compiler_params=pltpu.CompilerParams(
            needs_layout_passes=False, use_tc_tiling_on_sc=False
        ),
        scratch_types=[
            pltpu.VMEM((SEQ, EMB), jnp.float32),
            pltpu.VMEM_SHARED((SEQ, EMB), jnp.float32),
            pltpu.VMEM((per_w,), jnp.int32),
            pltpu.VMEM((NB, CH, EMB), jnp.float32),
            pltpu.SemaphoreType.DMA((NB,)),
            pltpu.SemaphoreType.DMA((NB,)),
            pltpu.SemaphoreType.DMA((NB,)),
        ],
    )
    def k(tok_hbm, table_hbm, pos_hbm, out_hbm,
          pos_v, pos_sh, idx_v, rows_v, gsem, wsem, psem):
        nc = info.num_cores
        sid = lax.axis_index("s")
        wid = sid * nc + lax.axis_index("c")
        wbase = wid * per_w

        pltpu.sync_copy(tok_hbm.at[pl.ds(wbase, per_w)], idx_v)
        pltpu.sync_copy(pos_hbm, pos_v)

        @pl.when(sid == 0)
        def _publish():
            pltpu.sync_copy(pos_v, pos_sh)

        plsc.subcore_barrier()

        def issue_prefill(b):
            pltpu.async_copy(pos_sh, rows_v.at[b], psem.at[b])

        def wait_prefill(b):
            pltpu.make_async_copy(pos_sh, rows_v.at[b], psem.at[b]).wait()

        def issue_gather(c, b):
            pltpu.async_copy(
                table_hbm.at[idx_v.at[pl.ds(c * CH, CH)]],
                rows_v.at[b],
                gsem.at[b],
                add=True,
            )

        def wait_gather(b):
            pltpu.make_async_copy(
                table_hbm.at[pl.ds(0, CH)], rows_v.at[b], gsem.at[b]
            ).wait()

        def issue_write(c, b):
            pltpu.async_copy(
                rows_v.at[b],
                out_hbm.at[pl.ds(wbase + c * CH, CH)],
                wsem.at[b],
            )

        def wait_write(b):
            pltpu.make_async_copy(
                rows_v.at[b], out_hbm.at[pl.ds(0, CH)], wsem.at[b]
            ).wait()

        for i in range(NB):
            issue_prefill(i)
        for i in range(AHEAD):
            wait_prefill(i)
            issue_gather(i, i)

        def chunk_body(c, _):
            b = c % NB
            wait_gather(b)

            cbase = c * CH

            def scan_body(g, acc):
                iv = idx_v[pl.ds(cbase + g * LANES, 16)]
                return acc + jnp.where(iv == PAD, jnp.int32(1), jnp.int32(0))

            acc = lax.fori_loop(
                0, CH // LANES, scan_body, jnp.zeros((16,), jnp.int32)
            )
            npad = jnp.sum(acc)

            @pl.when(npad > 0)
            def _fix():
                def fix_body(j, _):
                    bc = plsc.load_gather(
                        idx_v, [jnp.full((16,), cbase + j, jnp.int32)]
                    )
                    is_pad = bc == PAD
                    for kk in range(KV):
                        sl = pl.ds(kk * LANES, 16)
                        v = rows_v[b, j, sl]
                        rows_v[b, j, sl] = jnp.where(is_pad, pos_v[j, sl], v)
                    return 0

                lax.fori_loop(0, CH, fix_body, 0)

            def mul_body(g, _):
                j0 = g * 4
                for dj in range(4):
                    for kk in range(KV):
                        sl = pl.ds(kk * LANES, 16)
                        rows_v[b, j0 + dj, sl] = rows_v[b, j0 + dj, sl] * SCALE
                return 0

            lax.fori_loop(0, CH // 4, mul_body, 0)

            issue_write(c, b)

            @pl.when(c >= 1)
            def _drain():
                wait_write((c - 1) % NB)

                @pl.when(c < n_chunks - (AHEAD + 1))
                def _pf():
                    issue_prefill((c - 1) % NB)

            @pl.when(c + AHEAD < n_chunks)
            def _next():
                wait_prefill((c - 2) % NB)
                issue_gather(c + AHEAD, (c - 2) % NB)

            return 0

        lax.fori_loop(0, n_chunks, chunk_body, 0)
        wait_write((n_chunks - 1) % NB)

    return k(tok_flat, word_table_scaled, pos_table)


def kernel(tok_ids, word_table, pos_table):
    B, S = tok_ids.shape
    tok_flat = tok_ids.reshape(-1).astype(jnp.int32)
    out = _sc_embed(tok_flat, word_table, pos_table)
    return out.reshape(B, S, EMB)


# speedup vs baseline: 1.0909x; 1.0909x over previous
import functools

import jax
import jax.numpy as jnp
from jax import lax
from jax.experimental import pallas as pl
from jax.experimental.pallas import tpu as pltpu
from jax.experimental.pallas import tpu_sc as plsc

SCALE = 0.7071067811865476
EMB = 64
SEQ = 200
PAD = 0
LANES = 16


def _sc_embed(tok_flat, word_table_scaled, pos_table):
    N = tok_flat.shape[0]
    info = plsc.get_sparse_core_info()
    NW = info.num_cores * info.num_subcores
    per_w = N // NW
    CH = SEQ
    NB = 6
    AHEAD = 4
    n_chunks = per_w // CH
    assert per_w % CH == 0
    KV = EMB // LANES

    mesh = plsc.VectorSubcoreMesh(core_axis_name="c", subcore_axis_name="s")

    @functools.partial(
        pl.kernel,
        out_type=jax.ShapeDtypeStruct((N, EMB), jnp.float32),
        mesh=mesh,
        compiler_params=pltpu.CompilerParams(
            needs_layout_passes=False, use_tc_tiling_on_sc=False
        ),
        scratch_types=[
            pltpu.VMEM((SEQ, EMB), jnp.float32),
            pltpu.VMEM_SHARED((SEQ, EMB), jnp.float32),
            pltpu.VMEM((per_w,), jnp.int32),
            pltpu.VMEM((NB, CH, EMB), jnp.float32),
            pltpu.SemaphoreType.DMA((NB,)),
            pltpu.SemaphoreType.DMA((NB,)),
            pltpu.SemaphoreType.DMA((NB,)),
        ],
    )
    def k(tok_hbm, table_hbm, pos_hbm, out_hbm,
          pos_v, pos_sh, idx_v, rows_v, gsem, wsem, psem):
        nc = info.num_cores
        sid = lax.axis_index("s")
        wid = sid * nc + lax.axis_index("c")
        wbase = wid * per_w

        pltpu.sync_copy(tok_hbm.at[pl.ds(wbase, per_w)], idx_v)
        pltpu.sync_copy(pos_hbm, pos_v)

        @pl.when(sid == 0)
        def _publish():
            pltpu.sync_copy(pos_v, pos_sh)

        plsc.subcore_barrier()

        def issue_prefill(b):
            pltpu.async_copy(pos_sh, rows_v.at[b], psem.at[b])

        def wait_prefill(b):
            pltpu.make_async_copy(pos_sh, rows_v.at[b], psem.at[b]).wait()

        def issue_gather(c, b):
            pltpu.async_copy(
                table_hbm.at[idx_v.at[pl.ds(c * CH, CH)]],
                rows_v.at[b],
                gsem.at[b],
                add=True,
            )

        def wait_gather(b):
            pltpu.make_async_copy(
                table_hbm.at[pl.ds(0, CH)], rows_v.at[b], gsem.at[b]
            ).wait()

        def issue_write(c, b):
            pltpu.async_copy(
                rows_v.at[b],
                out_hbm.at[pl.ds(wbase + c * CH, CH)],
                wsem.at[b],
            )

        def wait_write(b):
            pltpu.make_async_copy(
                rows_v.at[b], out_hbm.at[pl.ds(0, CH)], wsem.at[b]
            ).wait()

        for i in range(NB):
            issue_prefill(i)
        for i in range(AHEAD):
            wait_prefill(i)
            issue_gather(i, i)

        def chunk_body(c, _):
            b = c % NB
            wait_gather(b)

            cbase = c * CH

            def scan_body(g, acc):
                iv = idx_v[pl.ds(cbase + g * LANES, 16)]
                return acc + jnp.where(iv == PAD, jnp.int32(1), jnp.int32(0))

            acc = lax.fori_loop(
                0, CH // LANES, scan_body, jnp.zeros((16,), jnp.int32)
            )
            npad = jnp.sum(acc)

            @pl.when(npad > 0)
            def _fix():
                def fix_body(j, _):
                    bc = plsc.load_gather(
                        idx_v, [jnp.full((16,), cbase + j, jnp.int32)]
                    )
                    is_pad = bc == PAD
                    for kk in range(KV):
                        sl = pl.ds(kk * LANES, 16)
                        v = rows_v[b, j, sl]
                        rows_v[b, j, sl] = jnp.where(is_pad, pos_v[j, sl], v)
                    return 0

                lax.fori_loop(0, CH, fix_body, 0)

            issue_write(c, b)

            @pl.when(c >= 1)
            def _drain():
                wait_write((c - 1) % NB)

                @pl.when(c < n_chunks - (AHEAD + 1))
                def _pf():
                    issue_prefill((c - 1) % NB)

            @pl.when(c + AHEAD < n_chunks)
            def _next():
                wait_prefill((c - 2) % NB)
                issue_gather(c + AHEAD, (c - 2) % NB)

            return 0

        lax.fori_loop(0, n_chunks, chunk_body, 0)
        wait_write((n_chunks - 1) % NB)

    return k(tok_flat, word_table_scaled, pos_table)


def kernel(tok_ids, word_table, pos_table):
    B, S = tok_ids.shape
    tok_flat = tok_ids.reshape(-1).astype(jnp.int32)
    sums = _sc_embed(tok_flat, word_table, pos_table)
    return (sums * jnp.float32(SCALE)).reshape(B, S, EMB)


# speedup vs baseline: 1.3199x; 1.2099x over previous
import functools

import jax
import jax.numpy as jnp
from jax import lax
from jax.experimental import pallas as pl
from jax.experimental.pallas import tpu as pltpu
from jax.experimental.pallas import tpu_sc as plsc

SCALE = 0.7071067811865476
EMB = 64
SEQ = 200
PAD = 0
LANES = 16


def _sc_embed(tok_flat, word_table_scaled, pos_table):
    N = tok_flat.shape[0]
    info = plsc.get_sparse_core_info()
    NW = info.num_cores * info.num_subcores
    per_w = N // NW
    CH = SEQ
    NB = 6
    AHEAD = 4
    n_chunks = per_w // CH
    assert per_w % CH == 0
    KV = EMB // LANES

    mesh = plsc.VectorSubcoreMesh(core_axis_name="c", subcore_axis_name="s")

    @functools.partial(
        pl.kernel,
        out_type=jax.ShapeDtypeStruct((N, EMB), jnp.float32),
        mesh=mesh,
        compiler_params=pltpu.CompilerParams(
            needs_layout_passes=False, use_tc_tiling_on_sc=False
        ),
        scratch_types=[
            pltpu.VMEM((SEQ, EMB), jnp.float32),
            pltpu.VMEM_SHARED((SEQ, EMB), jnp.float32),
            pltpu.VMEM((per_w,), jnp.int32),
            pltpu.VMEM((NB, CH, EMB), jnp.float32),
            pltpu.SemaphoreType.DMA((NB,)),
            pltpu.SemaphoreType.DMA((NB,)),
            pltpu.SemaphoreType.DMA((NB,)),
        ],
    )
    def k(tok_hbm, table_hbm, pos_hbm, out_hbm,
          pos_v, pos_sh, idx_v, rows_v, gsem, wsem, psem):
        nc = info.num_cores
        sid = lax.axis_index("s")
        wid = sid * nc + lax.axis_index("c")
        wbase = wid * per_w

        pltpu.sync_copy(tok_hbm.at[pl.ds(wbase, per_w)], idx_v)
        pltpu.sync_copy(pos_hbm, pos_v)

        @pl.when(sid == 0)
        def _publish():
            pltpu.sync_copy(pos_v, pos_sh)

        plsc.subcore_barrier()

        def issue_prefill(b):
            pltpu.async_copy(pos_sh, rows_v.at[b], psem.at[b])

        def wait_prefill(b):
            pltpu.make_async_copy(pos_sh, rows_v.at[b], psem.at[b]).wait()

        def issue_gather(c, b):
            pltpu.async_copy(
                table_hbm.at[idx_v.at[pl.ds(c * CH, CH)]],
                rows_v.at[b],
                gsem.at[b],
                add=True,
            )

        def wait_gather(b):
            pltpu.make_async_copy(
                table_hbm.at[pl.ds(0, CH)], rows_v.at[b], gsem.at[b]
            ).wait()

        def issue_write(c, b):
            pltpu.async_copy(
                rows_v.at[b],
                out_hbm.at[pl.ds(wbase + c * CH, CH)],
                wsem.at[b],
            )

        def wait_write(b):
            pltpu.make_async_copy(
                rows_v.at[b], out_hbm.at[pl.ds(0, CH)], wsem.at[b]
            ).wait()

        for i in range(NB):
            issue_prefill(i)
        for i in range(AHEAD):
            wait_prefill(i)
            issue_gather(i, i)

        def chunk_body(c, _):
            b = c % NB
            wait_gather(b)

            cbase = c * CH

            def scan_body(g, acc):
                iv = idx_v[pl.ds(cbase + g * LANES, 16)]
                return acc + jnp.where(iv == PAD, jnp.int32(1), jnp.int32(0))

            acc = lax.fori_loop(
                0, CH // LANES, scan_body, jnp.zeros((16,), jnp.int32)
            )
            npad = jnp.sum(acc)

            @pl.when(npad > 0)
            def _fix():
                def fix_body(j, _):
                    bc = plsc.load_gather(
                        idx_v, [jnp.full((16,), cbase + j, jnp.int32)]
                    )
                    is_pad = bc == PAD
                    for kk in range(KV):
                        sl = pl.ds(kk * LANES, 16)
                        v = rows_v[b, j, sl]
                        rows_v[b, j, sl] = jnp.where(is_pad, pos_v[j, sl], v)
                    return 0

                lax.fori_loop(0, CH, fix_body, 0)

            @plsc.parallel_loop(0, CH, step=1, unroll=8)
            def _mul(j):
                for kk in range(KV):
                    sl = pl.ds(kk * LANES, 16)
                    rows_v[b, j, sl] = rows_v[b, j, sl] * SCALE

            issue_write(c, b)

            @pl.when(c >= 1)
            def _drain():
                wait_write((c - 1) % NB)

                @pl.when(c < n_chunks - (AHEAD + 1))
                def _pf():
                    issue_prefill((c - 1) % NB)

            @pl.when(c + AHEAD < n_chunks)
            def _next():
                wait_prefill((c - 2) % NB)
                issue_gather(c + AHEAD, (c - 2) % NB)

            return 0

        lax.fori_loop(0, n_chunks, chunk_body, 0)
        wait_write((n_chunks - 1) % NB)

    return k(tok_flat, word_table_scaled, pos_table)


def kernel(tok_ids, word_table, pos_table):
    B, S = tok_ids.shape
    tok_flat = tok_ids.reshape(-1).astype(jnp.int32)
    out = _sc_embed(tok_flat, word_table, pos_table)
    return out.reshape(B, S, EMB)


# speedup vs baseline: 1.3212x; 1.0010x over previous
import functools

import jax
import jax.numpy as jnp
from jax import lax
from jax.experimental import pallas as pl
from jax.experimental.pallas import tpu as pltpu
from jax.experimental.pallas import tpu_sc as plsc

SCALE = 0.7071067811865476
EMB = 64
SEQ = 200
PAD = 0
LANES = 16


def _sc_embed(tok_flat, word_table, pos_table):
    N = tok_flat.shape[0]
    info = plsc.get_sparse_core_info()
    NW = info.num_cores * info.num_subcores
    per_w = N // NW
    CH = SEQ
    NB = 6
    AHEAD = 4
    n_chunks = per_w // CH
    assert per_w % CH == 0
    KV = EMB // LANES

    mesh = plsc.VectorSubcoreMesh(core_axis_name="c", subcore_axis_name="s")

    @functools.partial(
        pl.kernel,
        out_type=jax.ShapeDtypeStruct((N, EMB), jnp.float32),
        mesh=mesh,
        compiler_params=pltpu.CompilerParams(
            needs_layout_passes=False, use_tc_tiling_on_sc=False
        ),
        scratch_types=[
            pltpu.VMEM((SEQ, EMB), jnp.float32),
            pltpu.VMEM_SHARED((SEQ, EMB), jnp.float32),
            pltpu.VMEM((per_w,), jnp.int32),
            pltpu.VMEM((NB, CH, EMB), jnp.float32),
            pltpu.SemaphoreType.DMA((NB,)),
            pltpu.SemaphoreType.DMA((NB,)),
            pltpu.SemaphoreType.DMA((NB,)),
        ],
    )
    def k(tok_hbm, table_hbm, pos_hbm, out_hbm,
          pos_v, pos_sh, idx_v, rows_v, gsem, wsem, psem):
        nc = info.num_cores
        sid = lax.axis_index("s")
        wid = sid * nc + lax.axis_index("c")
        wbase = wid * per_w

        pltpu.sync_copy(tok_hbm.at[pl.ds(wbase, per_w)], idx_v)
        pltpu.sync_copy(pos_hbm, pos_v)

        @pl.when(sid == 0)
        def _publish():
            pltpu.sync_copy(pos_v, pos_sh)

        plsc.subcore_barrier()

        def issue_prefill(b):
            pltpu.async_copy(pos_sh, rows_v.at[b], psem.at[b])

        def wait_prefill(b):
            pltpu.make_async_copy(pos_sh, rows_v.at[b], psem.at[b]).wait()

        def issue_gather(c, b):
            pltpu.async_copy(
                table_hbm.at[idx_v.at[pl.ds(c * CH, CH)]],
                rows_v.at[b],
                gsem.at[b],
                add=True,
            )

        def wait_gather(b):
            pltpu.make_async_copy(
                table_hbm.at[pl.ds(0, CH)], rows_v.at[b], gsem.at[b]
            ).wait()

        def issue_write(c, b):
            pltpu.async_copy(
                rows_v.at[b],
                out_hbm.at[pl.ds(wbase + c * CH, CH)],
                wsem.at[b],
            )

        def wait_write(b):
            pltpu.make_async_copy(
                rows_v.at[b], out_hbm.at[pl.ds(0, CH)], wsem.at[b]
            ).wait()

        for i in range(NB):
            issue_prefill(i)
        for i in range(AHEAD):
            wait_prefill(i)
            issue_gather(i, i)

        def chunk_body(c, _):
            b = c % NB
            wait_gather(b)

            cbase = c * CH

            def scan_body(g, acc):
                iv = idx_v[pl.ds(cbase + g * LANES, 16)]
                return acc + jnp.where(iv == PAD, jnp.int32(1), jnp.int32(0))

            acc = lax.fori_loop(
                0, CH // LANES, scan_body, jnp.zeros((16,), jnp.int32)
            )
            npad = jnp.sum(acc)

            @pl.when(npad > 0)
            def _fix():
                def fix_body(j, _):
                    bc = plsc.load_gather(
                        idx_v, [jnp.full((16,), cbase + j, jnp.int32)]
                    )
                    is_pad = bc == PAD
                    for kk in range(KV):
                        sl = pl.ds(kk * LANES, 16)
                        v = rows_v[b, j, sl]
                        rows_v[b, j, sl] = jnp.where(is_pad, pos_v[j, sl], v)
                    return 0

                lax.fori_loop(0, CH, fix_body, 0)

            @plsc.parallel_loop(0, CH, step=1, unroll=8)
            def _mul(j):
                for kk in range(KV):
                    sl = pl.ds(kk * LANES, 16)
                    rows_v[b, j, sl] = rows_v[b, j, sl] * SCALE

            issue_write(c, b)

            @pl.when(c >= 1)
            def _drain():
                wait_write((c - 1) % NB)

                @pl.when(c < n_chunks - (AHEAD + 1))
                def _pf():
                    issue_prefill((c - 1) % NB)

            @pl.when(c + AHEAD < n_chunks)
            def _next():
                wait_prefill((c - 2) % NB)
                issue_gather(c + AHEAD, (c - 2) % NB)

            return 0

        lax.fori_loop(0, n_chunks, chunk_body, 0)
        wait_write((n_chunks - 1) % NB)

    return k(tok_flat, word_table, pos_table)


def kernel(tok_ids, word_table, pos_table):
    B, S = tok_ids.shape
    tok_flat = tok_ids.reshape(-1).astype(jnp.int32)
    out = _sc_embed(tok_flat, word_table, pos_table)
    return out.reshape(B, S, EMB)
